# baseline (speedup 1.0000x reference)
import jax
import jax.numpy as jnp
import numpy as np
from jax.experimental import pallas as pl
from jax.experimental.pallas import tpu as pltpu

_BS = 2048

_PATTERN = np.array([0, 1, 0, 1, 0, 1,
                     0, 1, 2, 0, 1, 2, 0, 1, 2,
                     0, 1, 2, 3] + [-1] * 6, dtype=np.float32)[:, None]


def _fused_body(xt_ref, pat_ref,
                e20_ref, e21_ref, e22_ref, e30_ref, e31_ref, e32_ref, e4_ref,
                W1_ref, b1_ref, W2_ref, b2_ref, W3_ref, b3_ref, out_ref):
    W1 = W1_ref[...]
    f32 = jnp.float32
    wfold = jnp.concatenate([
        jnp.dot(e20_ref[...], W1[0:4], preferred_element_type=f32),
        jnp.dot(e21_ref[...], W1[4:8], preferred_element_type=f32),
        jnp.dot(e22_ref[...], W1[8:12], preferred_element_type=f32),
        jnp.dot(e30_ref[...], W1[12:18], preferred_element_type=f32),
        jnp.dot(e31_ref[...], W1[18:24], preferred_element_type=f32),
        jnp.dot(e32_ref[...], W1[24:30], preferred_element_type=f32),
        jnp.dot(e4_ref[...], W1[30:38], preferred_element_type=f32),
        W1[38:44],
    ], axis=0)

    xt = xt_ref[...]
    row = jax.lax.broadcasted_iota(jnp.int32, xt.shape, 0)
    xt = jnp.where(row < 19, (xt == pat_ref[...]).astype(f32), xt)

    h = jax.lax.dot_general(xt, wfold, (((0,), (0,)), ((), ())),
                            preferred_element_type=f32)
    h = jnp.tanh(h + b1_ref[...])
    h = jnp.tanh(jnp.dot(h, W2_ref[...], preferred_element_type=f32)
                 + b2_ref[...])
    y = jnp.dot(h, W3_ref[...], preferred_element_type=f32) + b3_ref[...]
    out_ref[...] = jax.nn.sigmoid(y)


def kernel(con_x, cat_2, cat_3, cat_4,
           emb2_0, emb2_1, emb2_2, emb3_0, emb3_1, emb3_2, emb4,
           W1, b1, W2, b2, W3, b3):
    xt_packed = jnp.concatenate([
        jnp.repeat(cat_2.astype(jnp.float32).T, 2, axis=0),
        jnp.repeat(cat_3.astype(jnp.float32).T, 3, axis=0),
        jnp.repeat(cat_4.astype(jnp.float32).T, 4, axis=0),
        con_x.T,
    ], axis=0)

    b1r = b1.reshape(1, -1)
    b2r = b2.reshape(1, -1)
    b3r = b3.reshape(1, -1)

    B = con_x.shape[0]
    grid = (B // _BS,)

    def full(shape):
        nd = len(shape)
        return pl.BlockSpec(shape, lambda i: (0,) * nd)

    out = pl.pallas_call(
        _fused_body,
        grid=grid,
        in_specs=[
            pl.BlockSpec((25, _BS), lambda i: (0, i)),
            pl.BlockSpec((25, 1), lambda i: (0, 0)),
            full(emb2_0.shape), full(emb2_1.shape), full(emb2_2.shape),
            full(emb3_0.shape), full(emb3_1.shape), full(emb3_2.shape),
            full(emb4.shape),
            full(W1.shape), full(b1r.shape),
            full(W2.shape), full(b2r.shape),
            full(W3.shape), full(b3r.shape),
        ],
        out_specs=pl.BlockSpec((_BS, 2), lambda i: (i, 0)),
        out_shape=jax.ShapeDtypeStruct((B, 2), jnp.float32),
        compiler_params=pltpu.CompilerParams(
            dimension_semantics=("arbitrary",),
        ),
    )(xt_packed, jnp.asarray(_PATTERN),
      emb2_0, emb2_1, emb2_2, emb3_0, emb3_1, emb3_2, emb4,
      W1, b1r, W2, b2r, W3, b3r)
    return out


# speedup vs baseline: 1.0502x; 1.0502x over previous
import jax
import jax.numpy as jnp
import numpy as np
from jax.experimental import pallas as pl
from jax.experimental.pallas import tpu as pltpu

_BS = 4096

_PATTERN = np.array([0, 1, 0, 1, 0, 1,
                     0, 1, 2, 0, 1, 2, 0, 1, 2,
                     0, 1, 2, 3] + [-1] * 6, dtype=np.float32)[:, None]


def _fused_body(xt_ref, pat_ref,
                e20_ref, e21_ref, e22_ref, e30_ref, e31_ref, e32_ref, e4_ref,
                W1_ref, b1_ref, W2_ref, b2_ref, W3_ref, b3_ref, out_ref):
    W1 = W1_ref[...]
    f32 = jnp.float32
    wfold = jnp.concatenate([
        jnp.dot(e20_ref[...], W1[0:4], preferred_element_type=f32),
        jnp.dot(e21_ref[...], W1[4:8], preferred_element_type=f32),
        jnp.dot(e22_ref[...], W1[8:12], preferred_element_type=f32),
        jnp.dot(e30_ref[...], W1[12:18], preferred_element_type=f32),
        jnp.dot(e31_ref[...], W1[18:24], preferred_element_type=f32),
        jnp.dot(e32_ref[...], W1[24:30], preferred_element_type=f32),
        jnp.dot(e4_ref[...], W1[30:38], preferred_element_type=f32),
        W1[38:44],
    ], axis=0)

    xt = xt_ref[...]
    row = jax.lax.broadcasted_iota(jnp.int32, xt.shape, 0)
    xt = jnp.where(row < 19, (xt == pat_ref[...]).astype(f32), xt)

    h = jax.lax.dot_general(xt, wfold, (((0,), (0,)), ((), ())),
                            preferred_element_type=f32)
    h = jnp.tanh(h + b1_ref[...])
    h = jnp.tanh(jnp.dot(h.astype(jnp.bfloat16), W2_ref[...].astype(jnp.bfloat16),
                         preferred_element_type=f32)
                 + b2_ref[...])
    y = jnp.dot(h, W3_ref[...], preferred_element_type=f32) + b3_ref[...]
    out_ref[...] = jax.nn.sigmoid(y)


def kernel(con_x, cat_2, cat_3, cat_4,
           emb2_0, emb2_1, emb2_2, emb3_0, emb3_1, emb3_2, emb4,
           W1, b1, W2, b2, W3, b3):
    xt_packed = jnp.concatenate([
        jnp.repeat(cat_2.astype(jnp.float32).T, 2, axis=0),
        jnp.repeat(cat_3.astype(jnp.float32).T, 3, axis=0),
        jnp.repeat(cat_4.astype(jnp.float32).T, 4, axis=0),
        con_x.T,
    ], axis=0)

    b1r = b1.reshape(1, -1)
    b2r = b2.reshape(1, -1)
    b3r = b3.reshape(1, -1)

    B = con_x.shape[0]
    grid = (B // _BS,)

    def full(shape):
        nd = len(shape)
        return pl.BlockSpec(shape, lambda i: (0,) * nd)

    out = pl.pallas_call(
        _fused_body,
        grid=grid,
        in_specs=[
            pl.BlockSpec((25, _BS), lambda i: (0, i)),
            pl.BlockSpec((25, 1), lambda i: (0, 0)),
            full(emb2_0.shape), full(emb2_1.shape), full(emb2_2.shape),
            full(emb3_0.shape), full(emb3_1.shape), full(emb3_2.shape),
            full(emb4.shape),
            full(W1.shape), full(b1r.shape),
            full(W2.shape), full(b2r.shape),
            full(W3.shape), full(b3r.shape),
        ],
        out_specs=pl.BlockSpec((_BS, 2), lambda i: (i, 0)),
        out_shape=jax.ShapeDtypeStruct((B, 2), jnp.float32),
        compiler_params=pltpu.CompilerParams(
            dimension_semantics=("arbitrary",),
        ),
    )(xt_packed, jnp.asarray(_PATTERN),
      emb2_0, emb2_1, emb2_2, emb3_0, emb3_1, emb3_2, emb4,
      W1, b1r, W2, b2r, W3, b3r)
    return out


# speedup vs baseline: 1.0878x; 1.0358x over previous
import jax
import jax.numpy as jnp
from jax.experimental import pallas as pl
from jax.experimental.pallas import tpu as pltpu

_BS = 4096


def _fused_body(xt_ref,
                e20_ref, e21_ref, e22_ref, e30_ref, e31_ref, e32_ref, e4_ref,
                W1_ref, b1_ref, W2_ref, b2_ref, W3_ref, b3_ref, out_ref):
    W1 = W1_ref[...]
    f32 = jnp.float32
    t20 = jnp.dot(e20_ref[...], W1[0:4], preferred_element_type=f32)
    t21 = jnp.dot(e21_ref[...], W1[4:8], preferred_element_type=f32)
    t22 = jnp.dot(e22_ref[...], W1[8:12], preferred_element_type=f32)
    t30 = jnp.dot(e30_ref[...], W1[12:18], preferred_element_type=f32)
    t31 = jnp.dot(e31_ref[...], W1[18:24], preferred_element_type=f32)
    t32 = jnp.dot(e32_ref[...], W1[24:30], preferred_element_type=f32)
    t4 = jnp.dot(e4_ref[...], W1[30:38], preferred_element_type=f32)

    wfold = jnp.concatenate([
        t20[1:2] - t20[0:1],
        t21[1:2] - t21[0:1],
        t22[1:2] - t22[0:1],
        t30[1:2] - t30[0:1],
        t31[1:2] - t31[0:1],
        t32[1:2] - t32[0:1],
        t4[1:2] - t4[0:1],
        t30[2:3] - t30[1:2],
        t31[2:3] - t31[1:2],
        t32[2:3] - t32[1:2],
        t4[2:3] - t4[1:2],
        t4[3:4] - t4[2:3],
        W1[38:44],
    ], axis=0)

    base = (b1_ref[...] + t20[0:1] + t21[0:1] + t22[0:1]
            + t30[0:1] + t31[0:1] + t32[0:1] + t4[0:1])

    xt = xt_ref[...]
    s = jnp.concatenate([
        (xt[0:7] >= 1.0).astype(f32),
        (xt[3:7] >= 2.0).astype(f32),
        (xt[6:7] >= 3.0).astype(f32),
        xt[7:13],
    ], axis=0)

    h = jax.lax.dot_general(s, wfold, (((0,), (0,)), ((), ())),
                            preferred_element_type=f32)
    h = jnp.tanh(h + base)
    h = jnp.tanh(jnp.dot(h, W2_ref[...], preferred_element_type=f32)
                 + b2_ref[...])
    y = jnp.dot(h, W3_ref[...], preferred_element_type=f32) + b3_ref[...]
    out_ref[...] = jax.nn.sigmoid(y)


def kernel(con_x, cat_2, cat_3, cat_4,
           emb2_0, emb2_1, emb2_2, emb3_0, emb3_1, emb3_2, emb4,
           W1, b1, W2, b2, W3, b3):
    xt_packed = jnp.concatenate([
        cat_2.astype(jnp.float32).T,
        cat_3.astype(jnp.float32).T,
        cat_4.astype(jnp.float32).T,
        con_x.T,
    ], axis=0)

    b1r = b1.reshape(1, -1)
    b2r = b2.reshape(1, -1)
    b3r = b3.reshape(1, -1)

    B = con_x.shape[0]
    grid = (B // _BS,)

    def full(shape):
        nd = len(shape)
        return pl.BlockSpec(shape, lambda i: (0,) * nd)

    out = pl.pallas_call(
        _fused_body,
        grid=grid,
        in_specs=[
            pl.BlockSpec((13, _BS), lambda i: (0, i)),
            full(emb2_0.shape), full(emb2_1.shape), full(emb2_2.shape),
            full(emb3_0.shape), full(emb3_1.shape), full(emb3_2.shape),
            full(emb4.shape),
            full(W1.shape), full(b1r.shape),
            full(W2.shape), full(b2r.shape),
            full(W3.shape), full(b3r.shape),
        ],
        out_specs=pl.BlockSpec((_BS, 2), lambda i: (i, 0)),
        out_shape=jax.ShapeDtypeStruct((B, 2), jnp.float32),
        compiler_params=pltpu.CompilerParams(
            dimension_semantics=("arbitrary",),
        ),
    )(xt_packed,
      emb2_0, emb2_1, emb2_2, emb3_0, emb3_1, emb3_2, emb4,
      W1, b1r, W2, b2r, W3, b3r)
    return out
